# Initial kernel scaffold; baseline (speedup 1.0000x reference)
import jax
import jax.numpy as jnp
from jax.experimental import pallas as pl


def kernel(x):
    raise NotImplementedError("write your pallas kernel here")


# speedup vs baseline: 1.2326x; 1.2326x over previous
import jax
import jax.numpy as jnp
from jax.experimental import pallas as pl
from jax.experimental.pallas import tpu as pltpu

_B, _N, _D = 8, 1024, 256
_K = 512
_N_ITERS = 10


def _kmeans_body(x_ref, labels_ref, centers_ref):
    x = x_ref[0]
    x16 = x.astype(jnp.bfloat16)
    x2 = jnp.sum(x * x, axis=1, keepdims=True)
    ones_row = jnp.ones((1, _D), jnp.float32)
    ones_col = jnp.ones((_N, 1), jnp.bfloat16)
    kiota = jax.lax.broadcasted_iota(jnp.int32, (_N, _K), 1)

    def assign(c):
        c2 = jax.lax.dot_general(ones_row, c * c,
                                 (((1,), (1,)), ((), ())),
                                 preferred_element_type=jnp.float32,
                                 precision=jax.lax.Precision.HIGHEST)
        xc = jax.lax.dot_general(x16, c.astype(jnp.bfloat16),
                                 (((1,), (1,)), ((), ())),
                                 preferred_element_type=jnp.float32)
        d = (x2 + c2) - 2.0 * xc
        dmin = jnp.min(d, axis=1, keepdims=True)
        labels = jnp.min(jnp.where(d <= dmin, kiota, _K), axis=1,
                         keepdims=True)
        return labels

    def body(_, c):
        labels = assign(c)
        onehot = (labels == kiota).astype(jnp.bfloat16)
        counts = jax.lax.dot_general(onehot, ones_col,
                                     (((0,), (0,)), ((), ())),
                                     preferred_element_type=jnp.float32)
        sums = jax.lax.dot_general(onehot, x16,
                                   (((0,), (0,)), ((), ())),
                                   preferred_element_type=jnp.float32)
        newc = sums / jnp.maximum(counts, 1.0)
        return jnp.where(counts > 0, newc, c)

    c = jax.lax.fori_loop(0, _N_ITERS, body, x[:_K, :])
    labels_ref[0] = assign(c)
    centers_ref[0] = c


def kernel(x):
    labels, centers = pl.pallas_call(
        _kmeans_body,
        grid=(_B,),
        in_specs=[pl.BlockSpec((1, _N, _D), lambda b: (b, 0, 0))],
        out_specs=[
            pl.BlockSpec((1, _N, 1), lambda b: (b, 0, 0)),
            pl.BlockSpec((1, _K, _D), lambda b: (b, 0, 0)),
        ],
        out_shape=[
            jax.ShapeDtypeStruct((_B, _N, 1), jnp.int32),
            jax.ShapeDtypeStruct((_B, _K, _D), jnp.float32),
        ],
        compiler_params=pltpu.CompilerParams(
            dimension_semantics=("arbitrary",),
        ),
    )(x)
    return labels.reshape(_B, _N), centers


# speedup vs baseline: 1.2920x; 1.0482x over previous
import jax
import jax.numpy as jnp
from jax.experimental import pallas as pl
from jax.experimental.pallas import tpu as pltpu

_B, _N, _D = 8, 1024, 256
_K = 512
_N_ITERS = 10


def _kmeans_body(x_ref, labels_ref, centers_ref):
    x = x_ref[0]
    x16 = x.astype(jnp.bfloat16)
    ones_row = jnp.ones((1, _D), jnp.float32)
    ones_col = jnp.ones((_N, 1), jnp.bfloat16)
    kiota = jax.lax.broadcasted_iota(jnp.int32, (_N, _K), 1)

    def assign(c):
        c2 = jax.lax.dot_general(ones_row, c * c,
                                 (((1,), (1,)), ((), ())),
                                 preferred_element_type=jnp.float32,
                                 precision=jax.lax.Precision.HIGHEST)
        xc = jax.lax.dot_general(x16, c.astype(jnp.bfloat16),
                                 (((1,), (1,)), ((), ())),
                                 preferred_element_type=jnp.float32)
        d = c2 - 2.0 * xc
        labels = jnp.argmin(d, axis=1, keepdims=True)
        return labels.astype(jnp.int32)

    def body(_, c):
        labels = assign(c)
        onehot = (labels == kiota).astype(jnp.bfloat16)
        counts = jax.lax.dot_general(onehot, ones_col,
                                     (((0,), (0,)), ((), ())),
                                     preferred_element_type=jnp.float32)
        sums = jax.lax.dot_general(onehot, x16,
                                   (((0,), (0,)), ((), ())),
                                   preferred_element_type=jnp.float32)
        newc = sums / jnp.maximum(counts, 1.0)
        return jnp.where(counts > 0, newc, c)

    c = jax.lax.fori_loop(0, _N_ITERS, body, x[:_K, :])
    labels_ref[0] = assign(c)
    centers_ref[0] = c


def kernel(x):
    labels, centers = pl.pallas_call(
        _kmeans_body,
        grid=(_B,),
        in_specs=[pl.BlockSpec((1, _N, _D), lambda b: (b, 0, 0))],
        out_specs=[
            pl.BlockSpec((1, _N, 1), lambda b: (b, 0, 0)),
            pl.BlockSpec((1, _K, _D), lambda b: (b, 0, 0)),
        ],
        out_shape=[
            jax.ShapeDtypeStruct((_B, _N, 1), jnp.int32),
            jax.ShapeDtypeStruct((_B, _K, _D), jnp.float32),
        ],
        compiler_params=pltpu.CompilerParams(
            dimension_semantics=("arbitrary",),
        ),
    )(x)
    return labels.reshape(_B, _N), centers


# speedup vs baseline: 2.0709x; 1.6029x over previous
import jax
import jax.numpy as jnp
from jax.experimental import pallas as pl
from jax.experimental.pallas import tpu as pltpu

_B, _N, _D = 8, 1024, 256
_K = 512
_N_ITERS = 10


def _kmeans_body(x_ref, labels_ref, centers_ref):
    x = x_ref[0]
    x16 = x.astype(jnp.bfloat16)
    ones_col = jnp.ones((_N, 1), jnp.bfloat16)
    kiota_col = jax.lax.broadcasted_iota(jnp.int32, (_K, _N), 0)

    def assign(c):
        c2 = jnp.sum(c * c, axis=1, keepdims=True)
        cx = jax.lax.dot_general(c.astype(jnp.bfloat16), x16,
                                 (((1,), (1,)), ((), ())),
                                 preferred_element_type=jnp.float32)
        d = c2 - 2.0 * cx
        labels = jnp.argmin(d, axis=0, keepdims=True)
        return labels.astype(jnp.int32)

    def body(_, c):
        labels = assign(c)
        onehot = (labels == kiota_col).astype(jnp.bfloat16)
        counts = jax.lax.dot_general(onehot, ones_col,
                                     (((1,), (0,)), ((), ())),
                                     preferred_element_type=jnp.float32)
        sums = jax.lax.dot_general(onehot, x16,
                                   (((1,), (0,)), ((), ())),
                                   preferred_element_type=jnp.float32)
        newc = sums / jnp.maximum(counts, 1.0)
        return jnp.where(counts > 0, newc, c)

    c = jax.lax.fori_loop(0, _N_ITERS, body, x[:_K, :])
    labels_ref[0] = assign(c)
    centers_ref[0] = c


def kernel(x):
    labels, centers = pl.pallas_call(
        _kmeans_body,
        grid=(_B,),
        in_specs=[pl.BlockSpec((1, _N, _D), lambda b: (b, 0, 0))],
        out_specs=[
            pl.BlockSpec((1, 1, _N), lambda b: (b, 0, 0)),
            pl.BlockSpec((1, _K, _D), lambda b: (b, 0, 0)),
        ],
        out_shape=[
            jax.ShapeDtypeStruct((_B, 1, _N), jnp.int32),
            jax.ShapeDtypeStruct((_B, _K, _D), jnp.float32),
        ],
        compiler_params=pltpu.CompilerParams(
            dimension_semantics=("arbitrary",),
        ),
    )(x)
    return labels.reshape(_B, _N), centers


# speedup vs baseline: 2.3344x; 1.1272x over previous
import jax
import jax.numpy as jnp
from jax.experimental import pallas as pl
from jax.experimental.pallas import tpu as pltpu

_B, _N, _D = 8, 1024, 256
_K = 512
_N_ITERS = 10
_BPP = 2


def _kmeans_body(x_ref, labels_ref, centers_ref):
    ones_col = jnp.ones((_N, 1), jnp.bfloat16)
    kiota_col = jax.lax.broadcasted_iota(jnp.int32, (_K, _N), 0)
    xs = [x_ref[i] for i in range(_BPP)]
    x16s = [x.astype(jnp.bfloat16) for x in xs]

    def assign(c, x16):
        c2 = jnp.sum(c * c, axis=1, keepdims=True)
        cx = jax.lax.dot_general((-2.0 * c).astype(jnp.bfloat16), x16,
                                 (((1,), (1,)), ((), ())),
                                 preferred_element_type=jnp.float32)
        d = c2 + cx
        labels = jnp.argmin(d, axis=0, keepdims=True)
        return labels.astype(jnp.int32)

    def step(c, x16):
        labels = assign(c, x16)
        onehot = (labels == kiota_col).astype(jnp.bfloat16)
        counts = jax.lax.dot_general(onehot, ones_col,
                                     (((1,), (0,)), ((), ())),
                                     preferred_element_type=jnp.float32)
        sums = jax.lax.dot_general(onehot, x16,
                                   (((1,), (0,)), ((), ())),
                                   preferred_element_type=jnp.float32)
        newc = sums / jnp.maximum(counts, 1.0)
        return jnp.where(counts > 0, newc, c)

    def body(_, cs):
        return tuple(step(c, x16) for c, x16 in zip(cs, x16s))

    cs = jax.lax.fori_loop(0, _N_ITERS, body,
                           tuple(x[:_K, :] for x in xs))
    for i in range(_BPP):
        labels_ref[i] = assign(cs[i], x16s[i])
        centers_ref[i] = cs[i]


def kernel(x):
    labels, centers = pl.pallas_call(
        _kmeans_body,
        grid=(_B // _BPP,),
        in_specs=[pl.BlockSpec((_BPP, _N, _D), lambda b: (b, 0, 0))],
        out_specs=[
            pl.BlockSpec((_BPP, 1, _N), lambda b: (b, 0, 0)),
            pl.BlockSpec((_BPP, _K, _D), lambda b: (b, 0, 0)),
        ],
        out_shape=[
            jax.ShapeDtypeStruct((_B, 1, _N), jnp.int32),
            jax.ShapeDtypeStruct((_B, _K, _D), jnp.float32),
        ],
        compiler_params=pltpu.CompilerParams(
            dimension_semantics=("arbitrary",),
        ),
    )(x)
    return labels.reshape(_B, _N), centers


# speedup vs baseline: 2.4632x; 1.0552x over previous
import jax
import jax.numpy as jnp
from jax.experimental import pallas as pl
from jax.experimental.pallas import tpu as pltpu

_B, _N, _D = 8, 1024, 256
_K = 512
_N_ITERS = 10
_BPP = 4


def _kmeans_body(x_ref, labels_ref, centers_ref):
    ones_col = jnp.ones((_N, 1), jnp.bfloat16)
    kiota_col = jax.lax.broadcasted_iota(jnp.int32, (_K, _N), 0)
    xs = [x_ref[i] for i in range(_BPP)]
    x16s = [x.astype(jnp.bfloat16) for x in xs]
    x16es = [jnp.concatenate([x16, ones_col], axis=1) for x16 in x16s]

    def assign(c, x16):
        c2 = jnp.sum(c * c, axis=1, keepdims=True)
        cx = jax.lax.dot_general((-2.0 * c).astype(jnp.bfloat16), x16,
                                 (((1,), (1,)), ((), ())),
                                 preferred_element_type=jnp.float32)
        d = c2 + cx
        labels = jnp.argmin(d, axis=0, keepdims=True)
        return labels.astype(jnp.int32)

    def step(c, x16, x16e):
        labels = assign(c, x16)
        onehot = (labels == kiota_col).astype(jnp.bfloat16)
        sums_cnt = jax.lax.dot_general(onehot, x16e,
                                       (((1,), (0,)), ((), ())),
                                       preferred_element_type=jnp.float32)
        sums = sums_cnt[:, :_D]
        counts = sums_cnt[:, _D:]
        newc = sums / jnp.maximum(counts, 1.0)
        return jnp.where(counts > 0, newc, c)

    def body(_, cs):
        return tuple(step(c, x16, x16e)
                     for c, x16, x16e in zip(cs, x16s, x16es))

    cs = jax.lax.fori_loop(0, _N_ITERS, body,
                           tuple(x[:_K, :] for x in xs))
    for i in range(_BPP):
        labels_ref[i] = assign(cs[i], x16s[i])
        centers_ref[i] = cs[i]


def kernel(x):
    labels, centers = pl.pallas_call(
        _kmeans_body,
        grid=(_B // _BPP,),
        in_specs=[pl.BlockSpec((_BPP, _N, _D), lambda b: (b, 0, 0))],
        out_specs=[
            pl.BlockSpec((_BPP, 1, _N), lambda b: (b, 0, 0)),
            pl.BlockSpec((_BPP, _K, _D), lambda b: (b, 0, 0)),
        ],
        out_shape=[
            jax.ShapeDtypeStruct((_B, 1, _N), jnp.int32),
            jax.ShapeDtypeStruct((_B, _K, _D), jnp.float32),
        ],
        compiler_params=pltpu.CompilerParams(
            dimension_semantics=("arbitrary",),
        ),
    )(x)
    return labels.reshape(_B, _N), centers
